# baseline (speedup 1.0000x reference)
import jax
import jax.numpy as jnp
from jax import lax
from jax.experimental import pallas as pl
from jax.experimental.pallas import tpu as pltpu
from jax.experimental.pallas import tpu_sc as plsc

X_SIZE = 496
Y_SIZE = 432
NCHANNELS = 64
NPILLARS = 12000
XY = X_SIZE * Y_SIZE
NC = 2
NS = 16
NW = NC * NS
CHUNK = 6784
SB = 128


def _make_sc_call(batch):
    mesh = plsc.VectorSubcoreMesh(core_axis_name="c", subcore_axis_name="s")

    @pl.kernel(
        out_type=jax.ShapeDtypeStruct((batch, NCHANNELS, XY), jnp.float32),
        mesh=mesh,
        compiler_params=pltpu.CompilerParams(
            needs_layout_passes=False, use_tc_tiling_on_sc=False),
        scratch_types=[
            pltpu.VMEM((NPILLARS,), jnp.int32),
            pltpu.VMEM((NPILLARS,), jnp.int32),
            pltpu.VMEM((CHUNK,), jnp.int32),
            pltpu.VMEM((CHUNK,), jnp.float32),
            pltpu.VMEM((SB, NCHANNELS), jnp.float32),
            pltpu.VMEM((NCHANNELS, SB), jnp.float32),
            pltpu.VMEM((16,), jnp.float32),
            pltpu.SemaphoreType.DMA,
        ],
    )
    def sc_scatter(feat_hbm, c0_hbm, c1_hbm, flag_hbm, out_hbm,
                   c0_v, c1_v, perm_v, valid_v, rows_v, tbuf_v, flag_v, sem):
        cid = lax.axis_index("c")
        sid = lax.axis_index("s")
        wid = sid * NC + cid
        base = wid * CHUNK
        valid = jnp.minimum(CHUNK, XY - base)
        n_sb = valid // SB
        lanes = lax.iota(jnp.int32, 16)
        zero16 = jnp.zeros((16,), jnp.float32)

        pltpu.sync_copy(flag_hbm, flag_v)
        flagv = flag_v[pl.ds(0, 16)]

        def batch_body(b, carry):
            b_off = b * NPILLARS
            pltpu.sync_copy(c0_hbm.at[pl.ds(b_off, NPILLARS)], c0_v)
            pltpu.sync_copy(c1_hbm.at[pl.ds(b_off, NPILLARS)], c1_v)

            @plsc.parallel_loop(0, CHUNK // 128, 1)
            def _(i):
                for j in range(8):
                    off = i * 128 + j * 16
                    perm_v[pl.ds(off, 16)] = b_off + off + lanes
                    valid_v[pl.ds(off, 16)] = zero16

            def scan_body(i, c):
                for j in range(10):
                    k = i * 10 + j
                    v0 = c0_v[pl.ds(k * 16, 16)]
                    v1 = c1_v[pl.ds(k * 16, 16)]
                    local = v0 * Y_SIZE + v1 - base
                    m = (local >= 0) & (local < valid)
                    safe = jnp.where(m, local, 0)
                    pid = b_off + k * 16 + lanes
                    plsc.store_scatter(perm_v, [safe], pid, mask=m)
                    plsc.store_scatter(valid_v, [safe], flagv, mask=m)
                return c

            lax.fori_loop(0, NPILLARS // 160, scan_body, 0)

            def sb_body(s, c):
                idx_slice = perm_v.at[pl.ds(s * SB, SB)]
                pltpu.async_copy(feat_hbm.at[idx_slice], rows_v, sem).wait()

                rows_idx = [j * 16 + lanes for j in range(SB // 16)]
                w = [valid_v[pl.ds(s * SB + j * 16, 16)]
                     for j in range(SB // 16)]

                @plsc.parallel_loop(0, NCHANNELS, 1, unroll=4)
                def _(ch):
                    col = jnp.full((16,), ch, jnp.int32)
                    for j in range(SB // 16):
                        vals = plsc.load_gather(rows_v, [rows_idx[j], col])
                        tbuf_v[ch, pl.ds(j * 16, 16)] = vals * w[j]

                pltpu.sync_copy(
                    tbuf_v, out_hbm.at[b, :, pl.ds(base + s * SB, SB)])
                return c

            lax.fori_loop(0, n_sb, sb_body, 0)
            return carry

        lax.fori_loop(0, batch, batch_body, 0)

    return sc_scatter


def kernel(input_feat, coords, batch_size):
    B = input_feat.shape[0]
    feat_flat = input_feat.reshape(B * NPILLARS, NCHANNELS)
    c0 = coords[:, :, 0].astype(jnp.int32).reshape(-1)
    c1 = coords[:, :, 1].astype(jnp.int32).reshape(-1)
    flag16 = jnp.broadcast_to(
        (jnp.asarray(batch_size) == B).astype(jnp.float32), (16,))
    out = _make_sc_call(B)(feat_flat, c0, c1, flag16)
    return out.reshape(B, NCHANNELS, X_SIZE, Y_SIZE)
